# baseline (speedup 1.0000x reference)
import jax
import jax.numpy as jnp
from jax import lax
from jax.experimental import pallas as pl
from jax.experimental.pallas import tpu as pltpu
from jax.experimental.pallas import tpu_sc as plsc

N = 10000
NNZ = 160000
E = 5000
D_IN = 256
D_HID = 256
HALF = 128

NT = 16
NC = 2
NW = NC * NT

E_PAD = 5120
N_PAD = 10112
E_PER_TILE = E_PAD // NT
NV_PER_TILE = N_PAD // NT

PAIRS_A = 64
ROWS_A = 2 * PAIRS_A
NCH_A = 80
PAIRS_PER_WID = PAIRS_A * NCH_A
NNZ_PAD = PAIRS_PER_WID * NW
DEPTH_A = 2
E2_PAD = 2 * E_PAD
XE2_PER_TILE = E2_PAD // NT

CHUNK_B = 32
NCH_B = NNZ_PAD // NT // CHUNK_B
DEPTH_B = 2
E_XB = 5056


def _mm_body(x_ref, w_ref, o_ref):
    o_ref[...] = jnp.dot(x_ref[...], w_ref[...],
                         preferred_element_type=jnp.float32)


def _matmul(X, W):
    return pl.pallas_call(
        _mm_body,
        grid=(5,),
        in_specs=[
            pl.BlockSpec((2000, D_IN), lambda i: (i, 0)),
            pl.BlockSpec((D_IN, D_HID), lambda i: (0, 0)),
        ],
        out_specs=pl.BlockSpec((2000, D_HID), lambda i: (i, 0)),
        out_shape=jax.ShapeDtypeStruct((N, D_HID), jnp.float32),
    )(X, W)


def _comb_body(p_ref, cnt_ref, dege_ref, o_ref):
    a = p_ref[0] + p_ref[1]
    r = a.reshape(1280, 2, HALF)
    s = dege_ref[...] / jnp.maximum(cnt_ref[...], 1.0)
    o_ref[0] = r[:, 0, :] * s
    o_ref[1] = r[:, 1, :] * s


def _combine_scale(xe_part, cnt_sum, degE_pad):
    return pl.pallas_call(
        _comb_body,
        grid=(4,),
        in_specs=[
            pl.BlockSpec((NC, 2560, HALF), lambda i: (0, i, 0)),
            pl.BlockSpec((1280, 1), lambda i: (i, 0)),
            pl.BlockSpec((1280, 1), lambda i: (i, 0)),
        ],
        out_specs=pl.BlockSpec((NC, 1280, HALF), lambda i: (0, i, 0)),
        out_shape=jax.ShapeDtypeStruct((NC, E_PAD, HALF), jnp.float32),
    )(xe_part, cnt_sum, degE_pad)


def _norm_body(xv_ref, dv_ref, o_ref):
    a = xv_ref[0] * dv_ref[...]
    b = xv_ref[1] * dv_ref[...]
    ss = (jnp.sum(a * a, axis=1, keepdims=True)
          + jnp.sum(b * b, axis=1, keepdims=True))
    rn = jnp.sqrt(ss)
    sc = jnp.where(rn > 0, 1.0 / rn, 0.0)
    o_ref[:, :HALF] = a * sc
    o_ref[:, HALF:] = b * sc


def _normalize(xv_split, degV):
    return pl.pallas_call(
        _norm_body,
        grid=(5,),
        in_specs=[
            pl.BlockSpec((NC, 2000, HALF), lambda i: (0, i, 0)),
            pl.BlockSpec((2000, 1), lambda i: (i, 0)),
        ],
        out_specs=pl.BlockSpec((2000, D_HID), lambda i: (i, 0)),
        out_shape=jax.ShapeDtypeStruct((N, D_HID), jnp.float32),
    )(xv_split, degV)


def _pipeline(depth, nch, src_ref, gi_ref, si_ref, gsel, ssel, scatter_fn,
              ring, gbuf, gsems, isems, si2_ref=None, ring2=None):
    def idx_copy(j, k):
        pltpu.async_copy(gi_ref.at[gsel, j], ring.at[k], isems[k])
        pltpu.async_copy(si_ref.at[ssel, j], ring.at[4 + k], isems[k])
        if si2_ref is not None:
            pltpu.async_copy(si2_ref.at[ssel, j], ring2.at[k], isems[k])

    def idx_wait(j, k):
        pltpu.make_async_copy(
            gi_ref.at[gsel, j], ring.at[k], isems[k]).wait()
        pltpu.make_async_copy(
            si_ref.at[ssel, j], ring.at[4 + k], isems[k]).wait()
        if si2_ref is not None:
            pltpu.make_async_copy(
                si2_ref.at[ssel, j], ring2.at[k], isems[k]).wait()

    def gather(k):
        pltpu.async_copy(src_ref.at[ring.at[k]], gbuf.at[k], gsems[k])

    def gather_wait(k):
        pltpu.make_async_copy(
            src_ref.at[ring.at[k]], gbuf.at[k], gsems[k]).wait()

    def scat(k):
        if si2_ref is None:
            scatter_fn(gbuf.at[k], ring.at[4 + k])
        else:
            scatter_fn(gbuf.at[k], ring.at[4 + k], ring2.at[k])

    for k in range(depth):
        idx_copy(k, k)
    for k in range(depth):
        idx_wait(k, k)
        gather(k)

    @pl.loop(0, nch // depth)
    def _body(i):
        j = i * depth
        for k in range(depth):
            jj = j + k
            gather_wait(k)
            scat(k)

            @pl.when(jj + depth < nch)
            def _refill():
                idx_copy(jj + depth, k)
                idx_wait(jj + depth, k)
                gather(k)


def _sca_body(xp_ref, vga_ref, ea_ref, ec_ref, zw_ref, z1_ref, ones_ref,
              xe_out, cnt_out,
              xe_sh, cnt_sh,
              ring, ring2, gbuf, cb_v, ones_v,
              gs0, gs1, is0, is1):
    c = lax.axis_index("c")
    sid = lax.axis_index("s")
    wid = c * NT + sid

    pltpu.sync_copy(ones_ref, ones_v)
    pltpu.sync_copy(zw_ref,
                    xe_sh.at[pl.ds(sid * XE2_PER_TILE, XE2_PER_TILE)])
    pltpu.sync_copy(z1_ref, cb_v)
    pltpu.sync_copy(cb_v,
                    cnt_sh.at[pl.ds(sid * E_PER_TILE, E_PER_TILE)])
    plsc.subcore_barrier()

    def scatter_a(buf, sidx, cidx):
        pltpu.sync_copy(buf, xe_sh.at[sidx], add=True)
        pltpu.sync_copy(ones_v, cnt_sh.at[cidx], add=True)

    with jax.named_scope("phase_a"):
        _pipeline(DEPTH_A, NCH_A, xp_ref, vga_ref, ea_ref, wid, wid,
                  scatter_a,
                  ring, gbuf, (gs0, gs1), (is0, is1),
                  si2_ref=ec_ref, ring2=ring2)
        plsc.subcore_barrier()

    base2 = sid * XE2_PER_TILE
    pltpu.sync_copy(xe_sh.at[pl.ds(base2, XE2_PER_TILE)],
                    xe_out.at[pl.ds(c * E2_PAD + base2, XE2_PER_TILE)])
    base = sid * E_PER_TILE
    pltpu.sync_copy(cnt_sh.at[pl.ds(base, E_PER_TILE)], cb_v)
    pltpu.sync_copy(cb_v, cnt_out.at[pl.ds(c * E_PAD + base, E_PER_TILE)])


def _sc_phase_a(xp2, vga, ea, ec, zeros_a, zeros_1, ones_c):
    mesh = plsc.VectorSubcoreMesh(core_axis_name="c", subcore_axis_name="s")
    f = pl.kernel(
        _sca_body,
        out_type=(jax.ShapeDtypeStruct((NC * E2_PAD, HALF), jnp.float32),
                  jax.ShapeDtypeStruct((NC * E_PAD,), jnp.float32)),
        mesh=mesh,
        scratch_types=[
            pltpu.VMEM_SHARED((E2_PAD, HALF), jnp.float32),
            pltpu.VMEM_SHARED((E_PAD,), jnp.float32),
            pltpu.VMEM((8, ROWS_A), jnp.int32),
            pltpu.VMEM((8, PAIRS_A), jnp.int32),
            pltpu.VMEM((DEPTH_A, ROWS_A, HALF), jnp.float32),
            pltpu.VMEM((E_PER_TILE,), jnp.float32),
            pltpu.VMEM((PAIRS_A,), jnp.float32),
        ] + [pltpu.SemaphoreType.DMA] * 4,
    )
    return f(xp2, vga, ea, ec, zeros_a, zeros_1, ones_c)


def _scb_body(xe_ref, eb_ref, vs_ref, zw_ref, out_ref,
              xv_sh, xe_sp,
              ring, gbuf,
              gs0, gs1, is0, is1):
    c = lax.axis_index("c")
    sid = lax.axis_index("s")

    pltpu.sync_copy(zw_ref,
                    xv_sh.at[pl.ds(sid * NV_PER_TILE, NV_PER_TILE)])

    @pl.when(sid < NT - 1)
    def _stage():
        pltpu.sync_copy(xe_ref.at[pl.ds(c * E_PAD + sid * 320, 320)],
                        xe_sp.at[pl.ds(sid * 320, 320)])

    @pl.when(sid == NT - 1)
    def _stage_last():
        pltpu.sync_copy(
            xe_ref.at[pl.ds(c * E_PAD + 4800, E_XB - 4800)],
            xe_sp.at[pl.ds(4800, E_XB - 4800)])

    plsc.subcore_barrier()

    def scatter_b(buf, sidx):
        pltpu.sync_copy(buf, xv_sh.at[sidx], add=True)

    with jax.named_scope("phase_b"):
        _pipeline(DEPTH_B, NCH_B, xe_sp, eb_ref, vs_ref, sid, sid,
                  scatter_b,
                  ring, gbuf, (gs0, gs1), (is0, is1))
        plsc.subcore_barrier()

    out0 = sid * NV_PER_TILE
    pltpu.sync_copy(xv_sh.at[pl.ds(out0, NV_PER_TILE)],
                    out_ref.at[pl.ds(c * N_PAD + out0, NV_PER_TILE)])


def _sc_phase_b(xe, eb, vs, zeros_w):
    mesh = plsc.VectorSubcoreMesh(core_axis_name="c", subcore_axis_name="s")
    f = pl.kernel(
        _scb_body,
        out_type=jax.ShapeDtypeStruct((NC * N_PAD, HALF), jnp.float32),
        mesh=mesh,
        scratch_types=[
            pltpu.VMEM_SHARED((N_PAD, HALF), jnp.float32),
            pltpu.VMEM_SHARED((E_XB, HALF), jnp.float32),
            pltpu.VMEM((8, CHUNK_B), jnp.int32),
            pltpu.VMEM((DEPTH_B, CHUNK_B, HALF), jnp.float32),
        ] + [pltpu.SemaphoreType.DMA] * 4,
    )
    return f(xe, eb, vs, zeros_w)


@jax.jit
def kernel(X, vertex, edges, W, degE, degV):
    xp = _matmul(X, W)
    xp2 = xp.reshape(2 * N, HALF)

    pad = NNZ_PAD - NNZ
    vg = jnp.concatenate([vertex, jnp.zeros((pad,), jnp.int32)])
    e_p = jnp.concatenate([edges, jnp.full((pad,), E, jnp.int32)])
    vs = jnp.concatenate([vertex, jnp.full((pad,), N, jnp.int32)])
    vga = jnp.stack([2 * vg, 2 * vg + 1],
                    axis=-1).reshape(NW, NCH_A, ROWS_A)
    ea = jnp.stack([2 * e_p, 2 * e_p + 1],
                   axis=-1).reshape(NW, NCH_A, ROWS_A)
    ec = e_p.reshape(NW, NCH_A, PAIRS_A)
    eb = e_p.reshape(NT, NCH_B, CHUNK_B)
    vs_t = vs.reshape(NT, NCH_B, CHUNK_B)

    degE_pad = jnp.concatenate(
        [degE, jnp.ones((E_PAD - E, 1), jnp.float32)])
    zeros_a = jnp.zeros((XE2_PER_TILE, HALF), jnp.float32)
    zeros_b = jnp.zeros((NV_PER_TILE, HALF), jnp.float32)
    zeros_1 = jnp.zeros((E_PER_TILE,), jnp.float32)
    ones_c = jnp.ones((PAIRS_A,), jnp.float32)

    xe_part, cnt_part = _sc_phase_a(xp2, vga, ea, ec,
                                    zeros_a, zeros_1, ones_c)
    cnt2 = cnt_part.reshape(NC, E_PAD)
    cnt_sum = (cnt2[0] + cnt2[1])[:, None]
    xe = _combine_scale(xe_part.reshape(NC, E2_PAD, HALF),
                        cnt_sum, degE_pad)
    xv_flat = _sc_phase_b(xe.reshape(NC * E_PAD, HALF), eb, vs_t, zeros_b)
    xv_split = xv_flat.reshape(NC, N_PAD, HALF)[:, :N]
    return _normalize(xv_split, degV)


# speedup vs baseline: 1.2819x; 1.2819x over previous
import jax
import jax.numpy as jnp
from jax import lax
from jax.experimental import pallas as pl
from jax.experimental.pallas import tpu as pltpu
from jax.experimental.pallas import tpu_sc as plsc

N = 10000
NNZ = 160000
E = 5000
D_IN = 256
D_HID = 256
HALF = 128

NT = 16
NC = 2
CHUNK = 128
NCH = 80
PAIRS_PER_TILE = CHUNK * NCH
NNZ_PAD = PAIRS_PER_TILE * NT

E_PAD = 5120
N_PAD = 10112
E_PER_TILE = E_PAD // NT
NV_PER_TILE = N_PAD // NT

DEPTH_A = 4
DEPTH_B = 2
CHUNK_B = 32
NCH_B = PAIRS_PER_TILE // CHUNK_B
E_XB = 5056


def _mm_body(x_ref, w_ref, o_ref):
    o_ref[0] = jnp.dot(x_ref[...], w_ref[...],
                       preferred_element_type=jnp.float32)


def _matmul_split(X, W):
    return pl.pallas_call(
        _mm_body,
        grid=(5, NC),
        in_specs=[
            pl.BlockSpec((2000, D_IN), lambda i, c: (i, 0)),
            pl.BlockSpec((D_IN, HALF), lambda i, c: (0, c)),
        ],
        out_specs=pl.BlockSpec((1, 2000, HALF), lambda i, c: (c, i, 0)),
        out_shape=jax.ShapeDtypeStruct((NC, N, HALF), jnp.float32),
    )(X, W)


def _norm_body(xv_ref, dv_ref, o_ref):
    a = xv_ref[0] * dv_ref[...]
    b = xv_ref[1] * dv_ref[...]
    ss = (jnp.sum(a * a, axis=1, keepdims=True)
          + jnp.sum(b * b, axis=1, keepdims=True))
    rn = jnp.sqrt(ss)
    sc = jnp.where(rn > 0, 1.0 / rn, 0.0)
    o_ref[:, :HALF] = a * sc
    o_ref[:, HALF:] = b * sc


def _normalize(xv_split, degV):
    return pl.pallas_call(
        _norm_body,
        grid=(5,),
        in_specs=[
            pl.BlockSpec((NC, 2000, HALF), lambda i: (0, i, 0)),
            pl.BlockSpec((2000, 1), lambda i: (i, 0)),
        ],
        out_specs=pl.BlockSpec((2000, D_HID), lambda i: (i, 0)),
        out_shape=jax.ShapeDtypeStruct((N, D_HID), jnp.float32),
    )(xv_split, degV)


def _pipeline(depth, nch, src_ref, gi_ref, si_ref, gsel, ssel, scatter_fn,
              ring, gbuf, gsems, isems):
    def idx_copy(j, k):
        pltpu.async_copy(gi_ref.at[gsel, j], ring.at[k], isems[k])
        pltpu.async_copy(si_ref.at[ssel, j], ring.at[4 + k], isems[k])

    def idx_wait(j, k):
        pltpu.make_async_copy(
            gi_ref.at[gsel, j], ring.at[k], isems[k]).wait()
        pltpu.make_async_copy(
            si_ref.at[ssel, j], ring.at[4 + k], isems[k]).wait()

    def gather(k):
        pltpu.async_copy(src_ref.at[ring.at[k]], gbuf.at[k], gsems[k])

    def gather_wait(k):
        pltpu.make_async_copy(
            src_ref.at[ring.at[k]], gbuf.at[k], gsems[k]).wait()

    for k in range(depth):
        idx_copy(k, k)
    for k in range(depth):
        idx_wait(k, k)
        gather(k)

    @pl.loop(0, nch // depth)
    def _body(i):
        j = i * depth
        for k in range(depth):
            jj = j + k
            gather_wait(k)
            scatter_fn(gbuf.at[k], ring.at[4 + k])

            @pl.when(jj + depth < nch)
            def _refill():
                idx_copy(jj + depth, k)
                idx_wait(jj + depth, k)
                gather(k)


def _sca_body(xp_ref, vga_ref, ea_ref, dege_ref, zw_ref, z1_ref, ones_ref,
              xe_out,
              xe_sh, cnt_sh,
              ring, gbuf, dc_v, scal_v, ones_v,
              gs0, gs1, gs2, gs3, is0, is1, is2, is3):
    c = lax.axis_index("c")
    sid = lax.axis_index("s")
    wid = c * NT + sid

    pltpu.sync_copy(ones_ref, ones_v)
    pltpu.sync_copy(zw_ref.at[pl.ds(0, E_PER_TILE)],
                    xe_sh.at[pl.ds(sid * E_PER_TILE, E_PER_TILE)])
    pltpu.sync_copy(z1_ref, dc_v)
    pltpu.sync_copy(dc_v,
                    cnt_sh.at[pl.ds(sid * E_PER_TILE, E_PER_TILE)])
    plsc.subcore_barrier()

    def scatter_a(buf, sidx):
        pltpu.sync_copy(buf, xe_sh.at[sidx], add=True)
        pltpu.sync_copy(ones_v, cnt_sh.at[sidx], add=True)

    with jax.named_scope("phase_a"):
        _pipeline(DEPTH_A, NCH, xp_ref, vga_ref, ea_ref, wid, sid,
                  scatter_a,
                  ring, gbuf, (gs0, gs1, gs2, gs3), (is0, is1, is2, is3))
        plsc.subcore_barrier()

    base = sid * E_PER_TILE
    pltpu.sync_copy(dege_ref.at[pl.ds(base, E_PER_TILE)], dc_v)
    sbuf = gbuf.at[0, pl.ds(0, 16)]

    with jax.named_scope("scale"):
        @pl.loop(0, E_PER_TILE // 16)
        def _scale(jj):
            row0 = base + jj * 16
            pltpu.sync_copy(xe_sh.at[pl.ds(row0, 16)], sbuf)
            pltpu.sync_copy(cnt_sh.at[pl.ds(row0, 16)], scal_v)
            cvec = scal_v[...]
            dvec = dc_v[pl.ds(jj * 16, 16)]
            svec = dvec / jnp.maximum(cvec, 1.0)
            for r in range(16):
                s = svec[r]
                for k in range(HALF // 16):
                    sbuf[r, pl.ds(k * 16, 16)] = (
                        sbuf[r, pl.ds(k * 16, 16)] * s)
            pltpu.sync_copy(sbuf, xe_sh.at[pl.ds(row0, 16)])

    pltpu.sync_copy(xe_sh.at[pl.ds(base, E_PER_TILE)],
                    xe_out.at[pl.ds(c * E_PAD + base, E_PER_TILE)])


def _sc_phase_a(xp_flat, vga, ea, degE_pad, zeros_w, zeros_1, ones_c):
    mesh = plsc.VectorSubcoreMesh(core_axis_name="c", subcore_axis_name="s")
    f = pl.kernel(
        _sca_body,
        out_type=jax.ShapeDtypeStruct((NC * E_PAD, HALF), jnp.float32),
        mesh=mesh,
        scratch_types=[
            pltpu.VMEM_SHARED((E_PAD, HALF), jnp.float32),
            pltpu.VMEM_SHARED((E_PAD,), jnp.float32),
            pltpu.VMEM((8, CHUNK), jnp.int32),
            pltpu.VMEM((DEPTH_A, CHUNK, HALF), jnp.float32),
            pltpu.VMEM((E_PER_TILE,), jnp.float32),
            pltpu.VMEM((16,), jnp.float32),
            pltpu.VMEM((CHUNK,), jnp.float32),
        ] + [pltpu.SemaphoreType.DMA] * 8,
    )
    return f(xp_flat, vga, ea, degE_pad, zeros_w, zeros_1, ones_c)


def _scb_body(xe_ref, eb_ref, vs_ref, zw_ref, out_ref,
              xv_sh, xe_sp,
              ring, gbuf,
              gs0, gs1, is0, is1):
    c = lax.axis_index("c")
    sid = lax.axis_index("s")

    pltpu.sync_copy(zw_ref,
                    xv_sh.at[pl.ds(sid * NV_PER_TILE, NV_PER_TILE)])

    @pl.when(sid < NT - 1)
    def _stage():
        pltpu.sync_copy(xe_ref.at[pl.ds(c * E_PAD + sid * 320, 320)],
                        xe_sp.at[pl.ds(sid * 320, 320)])

    @pl.when(sid == NT - 1)
    def _stage_last():
        pltpu.sync_copy(
            xe_ref.at[pl.ds(c * E_PAD + 4800, E_XB - 4800)],
            xe_sp.at[pl.ds(4800, E_XB - 4800)])

    plsc.subcore_barrier()

    def scatter_b(buf, sidx):
        pltpu.sync_copy(buf, xv_sh.at[sidx], add=True)

    with jax.named_scope("phase_b"):
        _pipeline(DEPTH_B, NCH_B, xe_sp, eb_ref, vs_ref, sid, sid,
                  scatter_b,
                  ring, gbuf, (gs0, gs1), (is0, is1))
        plsc.subcore_barrier()

    out0 = sid * NV_PER_TILE
    pltpu.sync_copy(xv_sh.at[pl.ds(out0, NV_PER_TILE)],
                    out_ref.at[pl.ds(c * N_PAD + out0, NV_PER_TILE)])


def _sc_phase_b(xe, eb, vs, zeros_b):
    mesh = plsc.VectorSubcoreMesh(core_axis_name="c", subcore_axis_name="s")
    f = pl.kernel(
        _scb_body,
        out_type=jax.ShapeDtypeStruct((NC * N_PAD, HALF), jnp.float32),
        mesh=mesh,
        scratch_types=[
            pltpu.VMEM_SHARED((N_PAD, HALF), jnp.float32),
            pltpu.VMEM_SHARED((E_XB, HALF), jnp.float32),
            pltpu.VMEM((8, CHUNK_B), jnp.int32),
            pltpu.VMEM((DEPTH_B, CHUNK_B, HALF), jnp.float32),
        ] + [pltpu.SemaphoreType.DMA] * 4,
    )
    return f(xe, eb, vs, zeros_b)


@jax.jit
def kernel(X, vertex, edges, W, degE, degV):
    xp = _matmul_split(X, W)
    xp_flat = xp.reshape(NC * N, HALF)

    pad = NNZ_PAD - NNZ
    vg = jnp.concatenate([vertex, jnp.zeros((pad,), jnp.int32)])
    e_p = jnp.concatenate([edges, jnp.full((pad,), E, jnp.int32)])
    vs = jnp.concatenate([vertex, jnp.full((pad,), N, jnp.int32)])
    vg_t = vg.reshape(NT, NCH, CHUNK)
    e_t = e_p.reshape(NT, NCH, CHUNK)
    vga = jnp.concatenate([vg_t, vg_t + N], axis=0)
    eb = e_p.reshape(NT, NCH_B, CHUNK_B)
    vs_t = vs.reshape(NT, NCH_B, CHUNK_B)

    degE_pad = jnp.concatenate(
        [degE[:, 0], jnp.ones((E_PAD - E,), jnp.float32)])
    zeros_b = jnp.zeros((NV_PER_TILE, HALF), jnp.float32)
    zeros_1 = jnp.zeros((E_PER_TILE,), jnp.float32)
    ones_c = jnp.ones((CHUNK,), jnp.float32)

    xe = _sc_phase_a(xp_flat, vga, e_t, degE_pad, zeros_b, zeros_1, ones_c)
    xv_flat = _sc_phase_b(xe, eb, vs_t, zeros_b)
    xv_split = xv_flat.reshape(NC, N_PAD, HALF)[:, :N]
    return _normalize(xv_split, degV)
